# baseline (speedup 1.0000x reference)
import jax
import jax.numpy as jnp
from jax import lax
from jax.experimental import pallas as pl
from jax.experimental.pallas import tpu as pltpu
from jax.experimental.pallas import tpu_sc as plsc

DIM = 128
BATCH = 16384

NUM_CORES = 2
NUM_SUBCORES = 16
LANES = 16
NUM_WORKERS = NUM_CORES * NUM_SUBCORES
BPW = BATCH // NUM_WORKERS
CHUNK = 32
NCHUNK = BPW // CHUNK
NGROUP = CHUNK // LANES


def _body(s_h, r_h, o_h, t_h, ere_h, eim_h, rre_h, rim_h, tre_h, tim_h,
          out_h,
          idx_s, idx_r, idx_o, idx_t,
          bufs0, bufs1, out_v, sem0, sem1):
    wid = lax.axis_index("s") * NUM_CORES + lax.axis_index("c")
    base = wid * BPW

    pltpu.sync_copy(s_h.at[pl.ds(base, BPW)], idx_s)
    pltpu.sync_copy(r_h.at[pl.ds(base, BPW)], idx_r)
    pltpu.sync_copy(o_h.at[pl.ds(base, BPW)], idx_o)
    pltpu.sync_copy(t_h.at[pl.ds(base, BPW)], idx_t)

    lane = lax.broadcasted_iota(jnp.int32, (LANES,), 0)
    bufs = (bufs0, bufs1)
    sems = (sem0, sem1)

    def copies(c, b):
        sl = pl.ds(c * CHUNK, CHUNK)
        sre_r, sim_r, rre_r, rim_r, ore_r, oim_r, tre_r, tim_r = bufs[b]
        sem = sems[b]
        return [
            pltpu.make_async_copy(ere_h.at[idx_s.at[sl]], sre_r, sem),
            pltpu.make_async_copy(eim_h.at[idx_s.at[sl]], sim_r, sem),
            pltpu.make_async_copy(rre_h.at[idx_r.at[sl]], rre_r, sem),
            pltpu.make_async_copy(rim_h.at[idx_r.at[sl]], rim_r, sem),
            pltpu.make_async_copy(ere_h.at[idx_o.at[sl]], ore_r, sem),
            pltpu.make_async_copy(eim_h.at[idx_o.at[sl]], oim_r, sem),
            pltpu.make_async_copy(tre_h.at[idx_t.at[sl]], tre_r, sem),
            pltpu.make_async_copy(tim_h.at[idx_t.at[sl]], tim_r, sem),
        ]

    def issue(c, b):
        for cp in copies(c, b):
            cp.start()

    def wait(c, b):
        for cp in copies(c, b):
            cp.wait()

    def compute(c, b):
        sre_r, sim_r, rre_r, rim_r, ore_r, oim_r, tre_r, tim_r = bufs[b]
        for g in range(NGROUP):
            elem = g * LANES + lane

            @plsc.parallel_loop(0, DIM, 1, unroll=16,
                                carry=jnp.zeros((LANES,), jnp.float32))
            def acc(d, acc_in):
                dv = jnp.bitwise_and(d + lane, DIM - 1)
                sre = plsc.load_gather(sre_r, [elem, dv])
                sim = plsc.load_gather(sim_r, [elem, dv])
                rre = plsc.load_gather(rre_r, [elem, dv])
                rim = plsc.load_gather(rim_r, [elem, dv])
                ore = plsc.load_gather(ore_r, [elem, dv])
                oim = plsc.load_gather(oim_r, [elem, dv])
                tre = plsc.load_gather(tre_r, [elem, dv])
                tim = plsc.load_gather(tim_r, [elem, dv])
                pre = sre * ore + sim * oim
                pim = sim * ore - sre * oim
                rtre = rre * tre - rim * tim
                rtim = rre * tim + rim * tre
                return acc_in + (pre * rtre - pim * rtim)
            out_v[pl.ds(c * CHUNK + g * LANES, LANES)] = acc

    issue(0, 0)
    issue(1, 1)

    def pair_body(cp, carry):
        for b in range(2):
            c = 2 * cp + b
            wait(c, b)
            compute(c, b)

            @pl.when(c + 2 < NCHUNK)
            def _():
                issue(c + 2, b)
        return carry

    lax.fori_loop(0, NCHUNK // 2, pair_body, 0)
    pltpu.sync_copy(out_v, out_h.at[pl.ds(base, BPW)])


@jax.jit
def _run(s_i, r_i, o_i, t_i, E_re, E_im, R_re, R_im, T_re, T_im):
    mesh = plsc.VectorSubcoreMesh(core_axis_name="c", subcore_axis_name="s",
                                  num_cores=NUM_CORES,
                                  num_subcores=NUM_SUBCORES)
    buf_set = tuple(pltpu.VMEM((CHUNK, DIM), jnp.float32) for _ in range(8))
    f = pl.kernel(
        _body,
        out_type=jax.ShapeDtypeStruct((BATCH,), jnp.float32),
        mesh=mesh,
        scratch_types=[
            pltpu.VMEM((BPW,), jnp.int32),
            pltpu.VMEM((BPW,), jnp.int32),
            pltpu.VMEM((BPW,), jnp.int32),
            pltpu.VMEM((BPW,), jnp.int32),
            buf_set,
            tuple(pltpu.VMEM((CHUNK, DIM), jnp.float32) for _ in range(8)),
            pltpu.VMEM((BPW,), jnp.float32),
            pltpu.SemaphoreType.DMA,
            pltpu.SemaphoreType.DMA,
        ],
        compiler_params=pltpu.CompilerParams(needs_layout_passes=False),
    )
    return f(s_i, r_i, o_i, t_i, E_re, E_im, R_re, R_im, T_re, T_im)


def kernel(s, r, o, t, E_re, E_im, R_re, R_im, T_re, T_im):
    s_i = jnp.asarray(s[:, 0], jnp.int32)
    r_i = jnp.asarray(r[:, 0], jnp.int32)
    o_i = jnp.asarray(o[:, 0], jnp.int32)
    t_i = jnp.asarray(t[:, 0, 0], jnp.int32)
    out = _run(s_i, r_i, o_i, t_i, E_re, E_im, R_re, R_im, T_re, T_im)
    return out.reshape(BATCH, 1)


# speedup vs baseline: 1.6783x; 1.6783x over previous
import jax
import jax.numpy as jnp
from jax import lax
from jax.experimental import pallas as pl
from jax.experimental.pallas import tpu as pltpu
from jax.experimental.pallas import tpu_sc as plsc

DIM = 128
BATCH = 16384

NUM_CORES = 2
NUM_SUBCORES = 16
LANES = 16
NUM_WORKERS = NUM_CORES * NUM_SUBCORES
BPW = BATCH // NUM_WORKERS
CHUNK = 32
NCHUNK = BPW // CHUNK
NGROUP = CHUNK // LANES


def _body(s_h, r_h, o_h, t_h, ere_h, eim_h, rre_h, rim_h, tre_h, tim_h,
          out_h,
          idx_s, idx_r, idx_o, idx_t,
          bufs0, bufs1, out_v, sem0, sem1):
    wid = lax.axis_index("s") * NUM_CORES + lax.axis_index("c")
    base = wid * BPW

    pltpu.sync_copy(s_h.at[pl.ds(base, BPW)], idx_s)
    pltpu.sync_copy(r_h.at[pl.ds(base, BPW)], idx_r)
    pltpu.sync_copy(o_h.at[pl.ds(base, BPW)], idx_o)
    pltpu.sync_copy(t_h.at[pl.ds(base, BPW)], idx_t)

    lane = lax.broadcasted_iota(jnp.int32, (LANES,), 0)
    bufs = (bufs0, bufs1)
    sems = (sem0, sem1)

    def copies(c, b):
        sl = pl.ds(c * CHUNK, CHUNK)
        sre_r, sim_r, rre_r, rim_r, ore_r, oim_r, tre_r, tim_r = bufs[b]
        sem = sems[b]
        return [
            pltpu.make_async_copy(ere_h.at[idx_s.at[sl]], sre_r, sem),
            pltpu.make_async_copy(eim_h.at[idx_s.at[sl]], sim_r, sem),
            pltpu.make_async_copy(rre_h.at[idx_r.at[sl]], rre_r, sem),
            pltpu.make_async_copy(rim_h.at[idx_r.at[sl]], rim_r, sem),
            pltpu.make_async_copy(ere_h.at[idx_o.at[sl]], ore_r, sem),
            pltpu.make_async_copy(eim_h.at[idx_o.at[sl]], oim_r, sem),
            pltpu.make_async_copy(tre_h.at[idx_t.at[sl]], tre_r, sem),
            pltpu.make_async_copy(tim_h.at[idx_t.at[sl]], tim_r, sem),
        ]

    def issue(c, b):
        for cp in copies(c, b):
            cp.start()

    def wait(c, b):
        for cp in copies(c, b):
            cp.wait()

    def compute(c, b):
        sre_r, sim_r, rre_r, rim_r, ore_r, oim_r, tre_r, tim_r = bufs[b]
        elems = [g * LANES + lane for g in range(NGROUP)]

        def dim_body(d, accs):
            dv = jnp.bitwise_and(d + lane, DIM - 1)
            new = []
            for g in range(NGROUP):
                elem = elems[g]
                sre = plsc.load_gather(sre_r, [elem, dv])
                sim = plsc.load_gather(sim_r, [elem, dv])
                rre = plsc.load_gather(rre_r, [elem, dv])
                rim = plsc.load_gather(rim_r, [elem, dv])
                ore = plsc.load_gather(ore_r, [elem, dv])
                oim = plsc.load_gather(oim_r, [elem, dv])
                tre = plsc.load_gather(tre_r, [elem, dv])
                tim = plsc.load_gather(tim_r, [elem, dv])
                pre = sre * ore + sim * oim
                pim = sim * ore - sre * oim
                rtre = rre * tre - rim * tim
                rtim = rre * tim + rim * tre
                new.append(accs[g] + (pre * rtre - pim * rtim))
            return tuple(new)

        accs = lax.fori_loop(
            0, DIM, dim_body,
            tuple(jnp.zeros((LANES,), jnp.float32) for _ in range(NGROUP)))
        for g in range(NGROUP):
            out_v[pl.ds(c * CHUNK + g * LANES, LANES)] = accs[g]

    issue(0, 0)
    issue(1, 1)

    def pair_body(cp, carry):
        for b in range(2):
            c = 2 * cp + b
            wait(c, b)
            compute(c, b)

            @pl.when(c + 2 < NCHUNK)
            def _():
                issue(c + 2, b)
        return carry

    lax.fori_loop(0, NCHUNK // 2, pair_body, 0)
    pltpu.sync_copy(out_v, out_h.at[pl.ds(base, BPW)])


@jax.jit
def _run(s_i, r_i, o_i, t_i, E_re, E_im, R_re, R_im, T_re, T_im):
    mesh = plsc.VectorSubcoreMesh(core_axis_name="c", subcore_axis_name="s",
                                  num_cores=NUM_CORES,
                                  num_subcores=NUM_SUBCORES)
    buf_set = tuple(pltpu.VMEM((CHUNK, DIM), jnp.float32) for _ in range(8))
    f = pl.kernel(
        _body,
        out_type=jax.ShapeDtypeStruct((BATCH,), jnp.float32),
        mesh=mesh,
        scratch_types=[
            pltpu.VMEM((BPW,), jnp.int32),
            pltpu.VMEM((BPW,), jnp.int32),
            pltpu.VMEM((BPW,), jnp.int32),
            pltpu.VMEM((BPW,), jnp.int32),
            buf_set,
            tuple(pltpu.VMEM((CHUNK, DIM), jnp.float32) for _ in range(8)),
            pltpu.VMEM((BPW,), jnp.float32),
            pltpu.SemaphoreType.DMA,
            pltpu.SemaphoreType.DMA,
        ],
        compiler_params=pltpu.CompilerParams(needs_layout_passes=False),
    )
    return f(s_i, r_i, o_i, t_i, E_re, E_im, R_re, R_im, T_re, T_im)


def kernel(s, r, o, t, E_re, E_im, R_re, R_im, T_re, T_im):
    s_i = jnp.asarray(s[:, 0], jnp.int32)
    r_i = jnp.asarray(r[:, 0], jnp.int32)
    o_i = jnp.asarray(o[:, 0], jnp.int32)
    t_i = jnp.asarray(t[:, 0, 0], jnp.int32)
    out = _run(s_i, r_i, o_i, t_i, E_re, E_im, R_re, R_im, T_re, T_im)
    return out.reshape(BATCH, 1)


# speedup vs baseline: 1.7493x; 1.0423x over previous
import jax
import jax.numpy as jnp
from jax import lax
from jax.experimental import pallas as pl
from jax.experimental.pallas import tpu as pltpu
from jax.experimental.pallas import tpu_sc as plsc

DIM = 128
BATCH = 16384

NUM_CORES = 2
NUM_SUBCORES = 16
LANES = 16
NUM_WORKERS = NUM_CORES * NUM_SUBCORES
BPW = BATCH // NUM_WORKERS
CHUNK = 32
NCHUNK = BPW // CHUNK
NGROUP = CHUNK // LANES


def _body(s_h, r_h, o_h, t_h, ere_h, eim_h, rre_h, rim_h, tre_h, tim_h,
          out_h,
          idx_s, idx_r, idx_o, idx_t,
          bufs0, bufs1, out_v, sem0, sem1):
    wid = lax.axis_index("s") * NUM_CORES + lax.axis_index("c")
    base = wid * BPW

    icps = [
        pltpu.make_async_copy(s_h.at[pl.ds(base, BPW)], idx_s, sem0),
        pltpu.make_async_copy(r_h.at[pl.ds(base, BPW)], idx_r, sem0),
        pltpu.make_async_copy(o_h.at[pl.ds(base, BPW)], idx_o, sem0),
        pltpu.make_async_copy(t_h.at[pl.ds(base, BPW)], idx_t, sem0),
    ]
    for cp in icps:
        cp.start()
    for cp in icps:
        cp.wait()

    lane = lax.broadcasted_iota(jnp.int32, (LANES,), 0)
    bufs = (bufs0, bufs1)
    sems = (sem0, sem1)

    def copies(c, b):
        sl = pl.ds(c * CHUNK, CHUNK)
        sre_r, sim_r, rre_r, rim_r, ore_r, oim_r, tre_r, tim_r = bufs[b]
        sem = sems[b]
        return [
            pltpu.make_async_copy(ere_h.at[idx_s.at[sl]], sre_r, sem),
            pltpu.make_async_copy(eim_h.at[idx_s.at[sl]], sim_r, sem),
            pltpu.make_async_copy(rre_h.at[idx_r.at[sl]], rre_r, sem),
            pltpu.make_async_copy(rim_h.at[idx_r.at[sl]], rim_r, sem),
            pltpu.make_async_copy(ere_h.at[idx_o.at[sl]], ore_r, sem),
            pltpu.make_async_copy(eim_h.at[idx_o.at[sl]], oim_r, sem),
            pltpu.make_async_copy(tre_h.at[idx_t.at[sl]], tre_r, sem),
            pltpu.make_async_copy(tim_h.at[idx_t.at[sl]], tim_r, sem),
        ]

    def issue(c, b):
        for cp in copies(c, b):
            cp.start()

    def wait(c, b):
        for cp in copies(c, b):
            cp.wait()

    def compute(c, b):
        sre_r, sim_r, rre_r, rim_r, ore_r, oim_r, tre_r, tim_r = bufs[b]
        elems = [g * LANES + lane for g in range(NGROUP)]

        def dim_body(d, accs):
            dv = jnp.bitwise_and(d + lane, DIM - 1)
            new = []
            for g in range(NGROUP):
                elem = elems[g]
                sre = plsc.load_gather(sre_r, [elem, dv])
                sim = plsc.load_gather(sim_r, [elem, dv])
                rre = plsc.load_gather(rre_r, [elem, dv])
                rim = plsc.load_gather(rim_r, [elem, dv])
                ore = plsc.load_gather(ore_r, [elem, dv])
                oim = plsc.load_gather(oim_r, [elem, dv])
                tre = plsc.load_gather(tre_r, [elem, dv])
                tim = plsc.load_gather(tim_r, [elem, dv])
                pre = sre * ore + sim * oim
                pim = sim * ore - sre * oim
                rtre = rre * tre - rim * tim
                rtim = rre * tim + rim * tre
                new.append(accs[g] + (pre * rtre - pim * rtim))
            return tuple(new)

        accs = lax.fori_loop(
            0, DIM, dim_body,
            tuple(jnp.zeros((LANES,), jnp.float32) for _ in range(NGROUP)))
        for g in range(NGROUP):
            out_v[pl.ds(c * CHUNK + g * LANES, LANES)] = accs[g]

    issue(0, 0)
    issue(1, 1)

    def pair_body(cp, carry):
        for b in range(2):
            c = 2 * cp + b
            wait(c, b)
            compute(c, b)

            @pl.when(c + 2 < NCHUNK)
            def _():
                issue(c + 2, b)
        return carry

    lax.fori_loop(0, NCHUNK // 2, pair_body, 0)
    pltpu.sync_copy(out_v, out_h.at[pl.ds(base, BPW)])


@jax.jit
def _run(s_i, r_i, o_i, t_i, E_re, E_im, R_re, R_im, T_re, T_im):
    mesh = plsc.VectorSubcoreMesh(core_axis_name="c", subcore_axis_name="s",
                                  num_cores=NUM_CORES,
                                  num_subcores=NUM_SUBCORES)
    buf_set = tuple(pltpu.VMEM((CHUNK, DIM), jnp.float32) for _ in range(8))
    f = pl.kernel(
        _body,
        out_type=jax.ShapeDtypeStruct((BATCH,), jnp.float32),
        mesh=mesh,
        scratch_types=[
            pltpu.VMEM((BPW,), jnp.int32),
            pltpu.VMEM((BPW,), jnp.int32),
            pltpu.VMEM((BPW,), jnp.int32),
            pltpu.VMEM((BPW,), jnp.int32),
            buf_set,
            tuple(pltpu.VMEM((CHUNK, DIM), jnp.float32) for _ in range(8)),
            pltpu.VMEM((BPW,), jnp.float32),
            pltpu.SemaphoreType.DMA,
            pltpu.SemaphoreType.DMA,
        ],
        compiler_params=pltpu.CompilerParams(needs_layout_passes=False),
    )
    return f(s_i, r_i, o_i, t_i, E_re, E_im, R_re, R_im, T_re, T_im)


def kernel(s, r, o, t, E_re, E_im, R_re, R_im, T_re, T_im):
    s_i = jnp.asarray(s[:, 0], jnp.int32)
    r_i = jnp.asarray(r[:, 0], jnp.int32)
    o_i = jnp.asarray(o[:, 0], jnp.int32)
    t_i = jnp.asarray(t[:, 0, 0], jnp.int32)
    out = _run(s_i, r_i, o_i, t_i, E_re, E_im, R_re, R_im, T_re, T_im)
    return out.reshape(BATCH, 1)
